# baseline (speedup 1.0000x reference)
import functools

import jax
import jax.numpy as jnp
from jax import lax
from jax.experimental import pallas as pl
from jax.experimental.pallas import tpu as pltpu
from jax.experimental.pallas import tpu_sc as plsc

N = 10000
E = 320000
IN_FEATS = 128
HIDDEN = 16
HEADS = 4
HD = HIDDEN * HEADS
N_CLASSES = 2
INV_SCALE = 1.0 / (HIDDEN ** 0.5)

NPAD = 10240
ZR = NPAD // 16
DUMMY = N
NW = 32
EPT = 10240
EPAD = NW * EPT
CHUNK = 128
NCHUNKS = EPT // CHUNK
ACC_W = 72
DEPTH = 2
BR = 512
GRID = NPAD // BR
QP = HD // 2
KVP = HD


def _prelu(x, a):
    return jnp.where(x >= 0, x, a * x)


def _edge_body(q_hbm, kv_hbm, src_hbm, dst_hbm, zeros_hbm, out_hbm,
               acc, srca, dsta,
               qv0, qv1, kvv0, kvv1, msgv0, msgv1,
               sq0, sq1, skv0, skv1, ssc0, ssc1):
    c = lax.axis_index("c")
    s = lax.axis_index("s")
    wid = c * 16 + s
    pltpu.sync_copy(zeros_hbm, acc.at[pl.ds(s * ZR, ZR)])
    pltpu.sync_copy(src_hbm.at[wid], srca)
    pltpu.sync_copy(dst_hbm.at[wid], dsta)
    plsc.subcore_barrier()

    lane = lax.iota(jnp.int32, 16)
    bufs = ((qv0, kvv0, sq0, skv0), (qv1, kvv1, sq1, skv1))
    msgs = ((msgv0, ssc0), (msgv1, ssc1))

    def issue(j, qb, kb, sq, skv):
        pltpu.async_copy(q_hbm.at[dsta.at[j]], qb, sq, priority=1)
        pltpu.async_copy(kv_hbm.at[srca.at[j]], kb, skv, priority=1)

    def wait(j, qb, kb, sq, skv):
        pltpu.make_async_copy(q_hbm.at[dsta.at[j]], qb, sq).wait()
        pltpu.make_async_copy(kv_hbm.at[srca.at[j]], kb, skv).wait()

    def unpk(x):
        return plsc.unpack(plsc.bitcast(x, jnp.bfloat16),
                           format=plsc.PackFormat.INTERLEAVED)

    def compute(qb, kb, mv):
        def group_body(g, _):
            rows = g * 16 + lane
            for h in range(HEADS):
                lacc = None
                for cc in range(HIDDEN // 2):
                    col = jnp.full((16,), h * (HIDDEN // 2) + cc, jnp.int32)
                    qa, qbv = unpk(plsc.load_gather(qb, [rows, col]))
                    ka, kbv = unpk(plsc.load_gather(kb, [rows, col]))
                    t = qa * ka + qbv * kbv
                    lacc = t if lacc is None else lacc + t
                w = jnp.exp(lacc)
                plsc.store_scatter(mv, [rows, jnp.full((16,), HD + h, jnp.int32)], w)
                for cc in range(HIDDEN // 2):
                    vcol = jnp.full((16,), QP + h * (HIDDEN // 2) + cc, jnp.int32)
                    va, vb = unpk(plsc.load_gather(kb, [rows, vcol]))
                    base = h * HIDDEN + 2 * cc
                    plsc.store_scatter(mv, [rows, jnp.full((16,), base, jnp.int32)], w * va)
                    plsc.store_scatter(mv, [rows, jnp.full((16,), base + 1, jnp.int32)], w * vb)
            return None

        lax.fori_loop(0, CHUNK // 16, group_body, None)

    for b in range(DEPTH):
        issue(b, *bufs[b])

    def ring_body(i, _):
        for b in range(DEPTH):
            j = DEPTH * i + b
            qb, kb, sq, skv = bufs[b]
            mv, ssc = msgs[b]
            wait(j, qb, kb, sq, skv)

            @pl.when(j >= 2)
            def _():
                pltpu.make_async_copy(mv, acc.at[dsta.at[j - 2]], ssc).wait()

            compute(qb, kb, mv)
            pltpu.async_copy(mv, acc.at[dsta.at[j]], ssc, add=True)

            @pl.when(j + DEPTH < NCHUNKS)
            def _():
                issue(j + DEPTH, qb, kb, sq, skv)
        return None

    lax.fori_loop(0, NCHUNKS // DEPTH, ring_body, None)
    pltpu.make_async_copy(msgs[0][0], acc.at[dsta.at[NCHUNKS - 2]],
                          msgs[0][1]).wait()
    pltpu.make_async_copy(msgs[1][0], acc.at[dsta.at[NCHUNKS - 1]],
                          msgs[1][1]).wait()
    plsc.subcore_barrier()
    pltpu.sync_copy(acc.at[pl.ds(s * ZR, ZR)], out_hbm.at[c, pl.ds(s * ZR, ZR)])


def _edge_pass(q_tab, kv_tab, src3, dst3, zeros_blk):
    mesh = plsc.VectorSubcoreMesh(core_axis_name="c", subcore_axis_name="s",
                                  num_cores=2, num_subcores=16)
    run = functools.partial(
        pl.kernel,
        out_type=jax.ShapeDtypeStruct((2, NPAD, ACC_W), jnp.float32),
        mesh=mesh,
        compiler_params=pltpu.CompilerParams(needs_layout_passes=False,
                                             use_tc_tiling_on_sc=False),
        scratch_types=[
            pltpu.VMEM_SHARED((NPAD, ACC_W), jnp.float32),
            pltpu.VMEM((NCHUNKS, CHUNK), jnp.int32),
            pltpu.VMEM((NCHUNKS, CHUNK), jnp.int32),
            pltpu.VMEM((CHUNK, QP), jnp.int32),
            pltpu.VMEM((CHUNK, QP), jnp.int32),
            pltpu.VMEM((CHUNK, KVP), jnp.int32),
            pltpu.VMEM((CHUNK, KVP), jnp.int32),
            pltpu.VMEM((CHUNK, ACC_W), jnp.float32),
            pltpu.VMEM((CHUNK, ACC_W), jnp.float32),
            pltpu.SemaphoreType.DMA,
            pltpu.SemaphoreType.DMA,
            pltpu.SemaphoreType.DMA,
            pltpu.SemaphoreType.DMA,
            pltpu.SemaphoreType.DMA,
            pltpu.SemaphoreType.DMA,
        ],
    )(_edge_body)
    return run(q_tab, kv_tab, src3, dst3, zeros_blk)


def _row_spec(cols):
    return pl.BlockSpec((BR, cols), lambda i: (i, 0))


def _full_spec(shape):
    nd = len(shape)
    return pl.BlockSpec(shape, lambda i: (0,) * nd)


def _tc1_body(x_ref, oh_ref, tbl_ref, wpn, bpn, wpl, bpl, bng, bnb, am,
              wm, bm, wq, bq, wk, bk, wv, bv, ws, bs,
              q_ref, kv_ref, skip_ref):
    x = x_ref[...]
    lbl = jnp.dot(oh_ref[...], tbl_ref[...], preferred_element_type=jnp.float32)
    fused = (jnp.dot(x, wpn[...], preferred_element_type=jnp.float32) + bpn[...]
             + jnp.dot(lbl, wpl[...], preferred_element_type=jnp.float32) + bpl[...])
    fused = fused * (bng[...] / jnp.sqrt(1.0 + 1e-5)) + bnb[...]
    fused = _prelu(fused, am[0, 0])
    fused = jnp.dot(fused, wm[...], preferred_element_type=jnp.float32) + bm[...]
    h = x + fused
    q_ref[...] = (jnp.dot(h, wq[...], preferred_element_type=jnp.float32)
                  + bq[...]) * INV_SCALE
    k = jnp.dot(h, wk[...], preferred_element_type=jnp.float32) + bk[...]
    v = jnp.dot(h, wv[...], preferred_element_type=jnp.float32) + bv[...]
    kv_ref[...] = jnp.concatenate([k, v], axis=-1)
    skip_ref[...] = jnp.dot(h, ws[...], preferred_element_type=jnp.float32) + bs[...]


def _combine(a0, a1, skip, wg, bg, expand, lng, lnb, aact):
    num = a0[:, 0:HD] + a1[:, 0:HD]
    s4 = a0[:, HD:HD + HEADS] + a1[:, HD:HD + HEADS]
    sb = jnp.dot(s4, expand, preferred_element_type=jnp.float32)
    out = num / (sb + 1e-16)
    wa = wg[0:HD, :] + wg[2 * HD:3 * HD, :]
    wb = wg[HD:2 * HD, :] - wg[2 * HD:3 * HD, :]
    gl = (jnp.dot(skip, wa, preferred_element_type=jnp.float32)
          + jnp.dot(out, wb, preferred_element_type=jnp.float32)) + bg
    g = jax.nn.sigmoid(gl[:, 0:1])
    merged = g * skip + (1.0 - g) * out
    mu = jnp.mean(merged, axis=-1, keepdims=True)
    var = jnp.mean((merged - mu) ** 2, axis=-1, keepdims=True)
    y = (merged - mu) * lax.rsqrt(var + 1e-5) * lng + lnb
    return _prelu(y, aact)


def _tc2_body(a0_ref, a1_ref, skip_ref, wg, bg, expand, lng, lnb, am,
              wq, bq, wk, bk, wv, bv, ws, bs,
              q_ref, kv_ref, skip_o_ref):
    h = _combine(a0_ref[...], a1_ref[...], skip_ref[...], wg[...], bg[0, 0],
                 expand[...], lng[...], lnb[...], am[0, 0])
    q_ref[...] = (jnp.dot(h, wq[...], preferred_element_type=jnp.float32)
                  + bq[...]) * INV_SCALE
    k = jnp.dot(h, wk[...], preferred_element_type=jnp.float32) + bk[...]
    v = jnp.dot(h, wv[...], preferred_element_type=jnp.float32) + bv[...]
    kv_ref[...] = jnp.concatenate([k, v], axis=-1)
    skip_o_ref[...] = jnp.dot(h, ws[...], preferred_element_type=jnp.float32) + bs[...]


def _tc3_body(a0_ref, a1_ref, skip_ref, wg, bg, expand, lng, lnb, am,
              w1, b1, bcg, bcb, aclf, w2, b2, z_ref):
    h = _combine(a0_ref[...], a1_ref[...], skip_ref[...], wg[...], bg[0, 0],
                 expand[...], lng[...], lnb[...], am[0, 0])
    z = jnp.dot(h, w1[...], preferred_element_type=jnp.float32) + b1[...]
    z = z * (bcg[...] / jnp.sqrt(1.0 + 1e-5)) + bcb[...]
    z = _prelu(z, aclf[0, 0])
    z_ref[...] = jnp.dot(z, w2[...], preferred_element_type=jnp.float32) + b2[...]


def kernel(x, edge_index, labels, params):
    p = params
    f32 = jnp.float32

    x_pad = jnp.zeros((NPAD, IN_FEATS), f32).at[:N].set(x)
    oh = jax.nn.one_hot(labels, 8, dtype=f32)
    oh_pad = jnp.zeros((NPAD, 8), f32).at[:N].set(oh)
    tbl8 = jnp.zeros((8, IN_FEATS), f32).at[:N_CLASSES + 1].set(p['label_emb'])

    src = edge_index[0].astype(jnp.int32)
    dst = edge_index[1].astype(jnp.int32)
    src_p = jnp.full((EPAD,), DUMMY, jnp.int32).at[:E].set(src) \
        .reshape(NW, NCHUNKS, CHUNK)
    dst_p = jnp.full((EPAD,), DUMMY, jnp.int32).at[:E].set(dst) \
        .reshape(NW, NCHUNKS, CHUNK)
    zeros_blk = jnp.zeros((ZR, ACC_W), f32)
    expand = (jnp.arange(HD, dtype=jnp.int32)[None, :] // HIDDEN
              == jnp.arange(HEADS, dtype=jnp.int32)[:, None]).astype(f32)

    def pack2(a):
        b = a.astype(jnp.bfloat16).reshape(a.shape[0], -1, 2)
        return jax.lax.bitcast_convert_type(b, jnp.int32)

    def row1(a):
        return a.reshape(1, -1)

    def scl(a):
        return a.reshape(1, 1)

    wg_pad = jnp.zeros((3 * HD, 8), f32).at[:, 0:1].set(p['conv0_Wgate'])
    wg1_pad = jnp.zeros((3 * HD, 8), f32).at[:, 0:1].set(p['conv1_Wgate'])
    w2_pad = jnp.zeros((HD, 8), f32).at[:, :N_CLASSES].set(p['W_clf2'])
    b2_pad = jnp.zeros((1, 8), f32).at[0, :N_CLASSES].set(p['b_clf2'])

    tc1 = pl.pallas_call(
        _tc1_body,
        grid=(GRID,),
        in_specs=[
            _row_spec(IN_FEATS), _row_spec(8), _full_spec((8, IN_FEATS)),
            _full_spec((IN_FEATS, HD)), _full_spec((1, HD)),
            _full_spec((IN_FEATS, HD)), _full_spec((1, HD)),
            _full_spec((1, HD)), _full_spec((1, HD)), _full_spec((1, 1)),
            _full_spec((HD, IN_FEATS)), _full_spec((1, IN_FEATS)),
            _full_spec((IN_FEATS, HD)), _full_spec((1, HD)),
            _full_spec((IN_FEATS, HD)), _full_spec((1, HD)),
            _full_spec((IN_FEATS, HD)), _full_spec((1, HD)),
            _full_spec((IN_FEATS, HD)), _full_spec((1, HD)),
        ],
        out_specs=[_row_spec(HD), _row_spec(2 * HD), _row_spec(HD)],
        out_shape=[
            jax.ShapeDtypeStruct((NPAD, HD), f32),
            jax.ShapeDtypeStruct((NPAD, 2 * HD), f32),
            jax.ShapeDtypeStruct((NPAD, HD), f32),
        ],
    )
    q0, kv0, skip0 = tc1(
        x_pad, oh_pad, tbl8,
        p['W_proj_num'], row1(p['b_proj_num']),
        p['W_proj_label'], row1(p['b_proj_label']),
        row1(p['bn_merge_g']), row1(p['bn_merge_b']), scl(p['a_merge']),
        p['W_merge'], row1(p['b_merge']),
        p['conv0_Wq'], row1(p['conv0_bq']),
        p['conv0_Wk'], row1(p['conv0_bk']),
        p['conv0_Wv'], row1(p['conv0_bv']),
        p['conv0_Wskip'], row1(p['conv0_bskip']),
    )

    acc0 = _edge_pass(pack2(q0), pack2(kv0), src_p, dst_p, zeros_blk)

    tc2 = pl.pallas_call(
        _tc2_body,
        grid=(GRID,),
        in_specs=[
            _row_spec(ACC_W), _row_spec(ACC_W), _row_spec(HD),
            _full_spec((3 * HD, 8)), _full_spec((1, 1)),
            _full_spec((HEADS, HD)),
            _full_spec((1, HD)), _full_spec((1, HD)), _full_spec((1, 1)),
            _full_spec((HD, HD)), _full_spec((1, HD)),
            _full_spec((HD, HD)), _full_spec((1, HD)),
            _full_spec((HD, HD)), _full_spec((1, HD)),
            _full_spec((HD, HD)), _full_spec((1, HD)),
        ],
        out_specs=[_row_spec(HD), _row_spec(2 * HD), _row_spec(HD)],
        out_shape=[
            jax.ShapeDtypeStruct((NPAD, HD), f32),
            jax.ShapeDtypeStruct((NPAD, 2 * HD), f32),
            jax.ShapeDtypeStruct((NPAD, HD), f32),
        ],
    )
    q1, kv1, skip1 = tc2(
        acc0[0], acc0[1], skip0,
        wg_pad, scl(p['conv0_bgate']), expand,
        row1(p['conv0_ln_g']), row1(p['conv0_ln_b']), scl(p['a_act']),
        p['conv1_Wq'], row1(p['conv1_bq']),
        p['conv1_Wk'], row1(p['conv1_bk']),
        p['conv1_Wv'], row1(p['conv1_bv']),
        p['conv1_Wskip'], row1(p['conv1_bskip']),
    )

    acc1 = _edge_pass(pack2(q1), pack2(kv1), src_p, dst_p, zeros_blk)

    tc3 = pl.pallas_call(
        _tc3_body,
        grid=(GRID,),
        in_specs=[
            _row_spec(ACC_W), _row_spec(ACC_W), _row_spec(HD),
            _full_spec((3 * HD, 8)), _full_spec((1, 1)),
            _full_spec((HEADS, HD)),
            _full_spec((1, HD)), _full_spec((1, HD)), _full_spec((1, 1)),
            _full_spec((HD, HD)), _full_spec((1, HD)),
            _full_spec((1, HD)), _full_spec((1, HD)), _full_spec((1, 1)),
            _full_spec((HD, 8)), _full_spec((1, 8)),
        ],
        out_specs=[_row_spec(8)],
        out_shape=[jax.ShapeDtypeStruct((NPAD, 8), f32)],
    )
    (z,) = tc3(
        acc1[0], acc1[1], skip1,
        wg1_pad, scl(p['conv1_bgate']), expand,
        row1(p['conv1_ln_g']), row1(p['conv1_ln_b']), scl(p['a_act']),
        p['W_clf1'], row1(p['b_clf1']),
        row1(p['bn_clf_g']), row1(p['bn_clf_b']), scl(p['a_clf']),
        w2_pad, b2_pad,
    )
    return z[:N, :N_CLASSES]


# speedup vs baseline: 1.0114x; 1.0114x over previous
import functools

import jax
import jax.numpy as jnp
from jax import lax
from jax.experimental import pallas as pl
from jax.experimental.pallas import tpu as pltpu
from jax.experimental.pallas import tpu_sc as plsc

N = 10000
E = 320000
IN_FEATS = 128
HIDDEN = 16
HEADS = 4
HD = HIDDEN * HEADS
N_CLASSES = 2
INV_SCALE = 1.0 / (HIDDEN ** 0.5)

NPAD = 10240
ZR = NPAD // 16
DUMMY = N
NW = 32
EPT = 10240
EPAD = NW * EPT
CHUNK = 128
NCHUNKS = EPT // CHUNK
ACC_W = 72
DEPTH = 2
BR = 512
GRID = NPAD // BR
QP = HD // 2
KVP = HD


def _prelu(x, a):
    return jnp.where(x >= 0, x, a * x)


def _pack_pairs(x):
    return x.astype(jnp.bfloat16)


def _bitcast_pairs(x):
    return jax.lax.bitcast_convert_type(
        x.reshape(x.shape[0], x.shape[1] // 2, 2), jnp.int32)


def _edge_body(q_hbm, kv_hbm, src_hbm, dst_hbm, zeros_hbm, out_hbm,
               acc, srca, dsta,
               qv0, qv1, kvv0, kvv1, msgv0, msgv1,
               sq0, sq1, skv0, skv1, ssc0, ssc1):
    c = lax.axis_index("c")
    s = lax.axis_index("s")
    wid = c * 16 + s
    pltpu.sync_copy(zeros_hbm, acc.at[pl.ds(s * ZR, ZR)])
    pltpu.sync_copy(src_hbm.at[wid], srca)
    pltpu.sync_copy(dst_hbm.at[wid], dsta)
    plsc.subcore_barrier()

    lane = lax.iota(jnp.int32, 16)
    bufs = ((qv0, kvv0, sq0, skv0), (qv1, kvv1, sq1, skv1))
    msgs = ((msgv0, ssc0), (msgv1, ssc1))

    def issue(j, qb, kb, sq, skv):
        pltpu.async_copy(q_hbm.at[dsta.at[j]], qb, sq)
        pltpu.async_copy(kv_hbm.at[srca.at[j]], kb, skv)

    def wait(j, qb, kb, sq, skv):
        pltpu.make_async_copy(q_hbm.at[dsta.at[j]], qb, sq).wait()
        pltpu.make_async_copy(kv_hbm.at[srca.at[j]], kb, skv).wait()

    def unpk(x):
        return plsc.unpack(plsc.bitcast(x, jnp.bfloat16),
                           format=plsc.PackFormat.INTERLEAVED)

    def compute(qb, kb, mv):
        def group_body(g, _):
            rows = g * 16 + lane
            for h in range(HEADS):
                lacc = None
                for cc in range(HIDDEN // 2):
                    col = jnp.full((16,), h * (HIDDEN // 2) + cc, jnp.int32)
                    qa, qbv = unpk(plsc.load_gather(qb, [rows, col]))
                    ka, kbv = unpk(plsc.load_gather(kb, [rows, col]))
                    t = qa * ka + qbv * kbv
                    lacc = t if lacc is None else lacc + t
                w = jnp.exp(lacc)
                plsc.store_scatter(mv, [rows, jnp.full((16,), HD + h, jnp.int32)], w)
                for cc in range(HIDDEN // 2):
                    vcol = jnp.full((16,), QP + h * (HIDDEN // 2) + cc, jnp.int32)
                    va, vb = unpk(plsc.load_gather(kb, [rows, vcol]))
                    base = h * HIDDEN + 2 * cc
                    plsc.store_scatter(mv, [rows, jnp.full((16,), base, jnp.int32)], w * va)
                    plsc.store_scatter(mv, [rows, jnp.full((16,), base + 1, jnp.int32)], w * vb)
            return None

        lax.fori_loop(0, CHUNK // 16, group_body, None)

    for b in range(DEPTH):
        issue(b, *bufs[b])

    def ring_body(i, _):
        for b in range(DEPTH):
            j = DEPTH * i + b
            qb, kb, sq, skv = bufs[b]
            mv, ssc = msgs[b]
            wait(j, qb, kb, sq, skv)

            @pl.when(j >= 2)
            def _():
                pltpu.make_async_copy(mv, acc.at[dsta.at[j - 2]], ssc).wait()

            compute(qb, kb, mv)
            pltpu.async_copy(mv, acc.at[dsta.at[j]], ssc, add=True)

            @pl.when(j + DEPTH < NCHUNKS)
            def _():
                issue(j + DEPTH, qb, kb, sq, skv)
        return None

    lax.fori_loop(0, NCHUNKS // DEPTH, ring_body, None)
    pltpu.make_async_copy(msgs[0][0], acc.at[dsta.at[NCHUNKS - 2]],
                          msgs[0][1]).wait()
    pltpu.make_async_copy(msgs[1][0], acc.at[dsta.at[NCHUNKS - 1]],
                          msgs[1][1]).wait()
    plsc.subcore_barrier()
    pltpu.sync_copy(acc.at[pl.ds(s * ZR, ZR)], out_hbm.at[c, pl.ds(s * ZR, ZR)])


def _edge_pass(q_tab, kv_tab, src3, dst3, zeros_blk):
    mesh = plsc.VectorSubcoreMesh(core_axis_name="c", subcore_axis_name="s",
                                  num_cores=2, num_subcores=16)
    run = functools.partial(
        pl.kernel,
        out_type=jax.ShapeDtypeStruct((2, NPAD, ACC_W), jnp.float32),
        mesh=mesh,
        compiler_params=pltpu.CompilerParams(needs_layout_passes=False,
                                             use_tc_tiling_on_sc=False),
        scratch_types=[
            pltpu.VMEM_SHARED((NPAD, ACC_W), jnp.float32),
            pltpu.VMEM((NCHUNKS, CHUNK), jnp.int32),
            pltpu.VMEM((NCHUNKS, CHUNK), jnp.int32),
            pltpu.VMEM((CHUNK, QP), jnp.int32),
            pltpu.VMEM((CHUNK, QP), jnp.int32),
            pltpu.VMEM((CHUNK, KVP), jnp.int32),
            pltpu.VMEM((CHUNK, KVP), jnp.int32),
            pltpu.VMEM((CHUNK, ACC_W), jnp.float32),
            pltpu.VMEM((CHUNK, ACC_W), jnp.float32),
            pltpu.SemaphoreType.DMA,
            pltpu.SemaphoreType.DMA,
            pltpu.SemaphoreType.DMA,
            pltpu.SemaphoreType.DMA,
            pltpu.SemaphoreType.DMA,
            pltpu.SemaphoreType.DMA,
        ],
    )(_edge_body)
    return run(q_tab, kv_tab, src3, dst3, zeros_blk)


def _row_spec(cols):
    return pl.BlockSpec((BR, cols), lambda i: (i, 0))


def _full_spec(shape):
    nd = len(shape)
    return pl.BlockSpec(shape, lambda i: (0,) * nd)


def _tc1_body(x_ref, oh_ref, tbl_ref, wpn, bpn, wpl, bpl, bng, bnb, am,
              wm, bm, wq, bq, wk, bk, wv, bv, ws, bs,
              q_ref, kv_ref, skip_ref):
    x = x_ref[...]
    lbl = jnp.dot(oh_ref[...], tbl_ref[...], preferred_element_type=jnp.float32)
    fused = (jnp.dot(x, wpn[...], preferred_element_type=jnp.float32) + bpn[...]
             + jnp.dot(lbl, wpl[...], preferred_element_type=jnp.float32) + bpl[...])
    fused = fused * (bng[...] / jnp.sqrt(1.0 + 1e-5)) + bnb[...]
    fused = _prelu(fused, am[0, 0])
    fused = jnp.dot(fused, wm[...], preferred_element_type=jnp.float32) + bm[...]
    h = x + fused
    q = (jnp.dot(h, wq[...], preferred_element_type=jnp.float32)
         + bq[...]) * INV_SCALE
    k = jnp.dot(h, wk[...], preferred_element_type=jnp.float32) + bk[...]
    v = jnp.dot(h, wv[...], preferred_element_type=jnp.float32) + bv[...]
    q_ref[...] = _pack_pairs(q)
    kv_ref[...] = _pack_pairs(jnp.concatenate([k, v], axis=-1))
    skip_ref[...] = jnp.dot(h, ws[...], preferred_element_type=jnp.float32) + bs[...]


def _combine(a0, a1, skip, wg, bg, expand, lng, lnb, aact):
    num = a0[:, 0:HD] + a1[:, 0:HD]
    s4 = a0[:, HD:HD + HEADS] + a1[:, HD:HD + HEADS]
    sb = jnp.dot(s4, expand, preferred_element_type=jnp.float32)
    out = num / (sb + 1e-16)
    wa = wg[0:HD, :] + wg[2 * HD:3 * HD, :]
    wb = wg[HD:2 * HD, :] - wg[2 * HD:3 * HD, :]
    gl = (jnp.dot(skip, wa, preferred_element_type=jnp.float32)
          + jnp.dot(out, wb, preferred_element_type=jnp.float32)) + bg
    g = jax.nn.sigmoid(gl[:, 0:1])
    merged = g * skip + (1.0 - g) * out
    mu = jnp.mean(merged, axis=-1, keepdims=True)
    var = jnp.mean((merged - mu) ** 2, axis=-1, keepdims=True)
    y = (merged - mu) * lax.rsqrt(var + 1e-5) * lng + lnb
    return _prelu(y, aact)


def _tc2_body(a0_ref, a1_ref, skip_ref, wg, bg, expand, lng, lnb, am,
              wq, bq, wk, bk, wv, bv, ws, bs,
              q_ref, kv_ref, skip_o_ref):
    h = _combine(a0_ref[...][0], a1_ref[...][0], skip_ref[...], wg[...],
                 bg[0, 0], expand[...], lng[...], lnb[...], am[0, 0])
    q = (jnp.dot(h, wq[...], preferred_element_type=jnp.float32)
         + bq[...]) * INV_SCALE
    k = jnp.dot(h, wk[...], preferred_element_type=jnp.float32) + bk[...]
    v = jnp.dot(h, wv[...], preferred_element_type=jnp.float32) + bv[...]
    q_ref[...] = _pack_pairs(q)
    kv_ref[...] = _pack_pairs(jnp.concatenate([k, v], axis=-1))
    skip_o_ref[...] = jnp.dot(h, ws[...], preferred_element_type=jnp.float32) + bs[...]


def _tc3_body(a0_ref, a1_ref, skip_ref, wg, bg, expand, lng, lnb, am,
              w1, b1, bcg, bcb, aclf, w2, b2, z_ref):
    h = _combine(a0_ref[...][0], a1_ref[...][0], skip_ref[...], wg[...],
                 bg[0, 0], expand[...], lng[...], lnb[...], am[0, 0])
    z = jnp.dot(h, w1[...], preferred_element_type=jnp.float32) + b1[...]
    z = z * (bcg[...] / jnp.sqrt(1.0 + 1e-5)) + bcb[...]
    z = _prelu(z, aclf[0, 0])
    z_ref[...] = jnp.dot(z, w2[...], preferred_element_type=jnp.float32) + b2[...]


def kernel(x, edge_index, labels, params):
    p = params
    f32 = jnp.float32

    x_pad = jnp.zeros((NPAD, IN_FEATS), f32).at[:N].set(x)
    oh = jax.nn.one_hot(labels, 8, dtype=f32)
    oh_pad = jnp.zeros((NPAD, 8), f32).at[:N].set(oh)
    tbl8 = jnp.zeros((8, IN_FEATS), f32).at[:N_CLASSES + 1].set(p['label_emb'])

    src = edge_index[0].astype(jnp.int32)
    dst = edge_index[1].astype(jnp.int32)
    src_p = jnp.full((EPAD,), DUMMY, jnp.int32).at[:E].set(src) \
        .reshape(NW, NCHUNKS, CHUNK)
    dst_p = jnp.full((EPAD,), DUMMY, jnp.int32).at[:E].set(dst) \
        .reshape(NW, NCHUNKS, CHUNK)
    zeros_blk = jnp.zeros((ZR, ACC_W), f32)
    expand = (jnp.arange(HD, dtype=jnp.int32)[None, :] // HIDDEN
              == jnp.arange(HEADS, dtype=jnp.int32)[:, None]).astype(f32)

    def row1(a):
        return a.reshape(1, -1)

    def scl(a):
        return a.reshape(1, 1)

    wg_pad = jnp.zeros((3 * HD, 8), f32).at[:, 0:1].set(p['conv0_Wgate'])
    wg1_pad = jnp.zeros((3 * HD, 8), f32).at[:, 0:1].set(p['conv1_Wgate'])
    w2_pad = jnp.zeros((HD, 8), f32).at[:, :N_CLASSES].set(p['W_clf2'])
    b2_pad = jnp.zeros((1, 8), f32).at[0, :N_CLASSES].set(p['b_clf2'])

    tc1 = pl.pallas_call(
        _tc1_body,
        grid=(GRID,),
        in_specs=[
            _row_spec(IN_FEATS), _row_spec(8), _full_spec((8, IN_FEATS)),
            _full_spec((IN_FEATS, HD)), _full_spec((1, HD)),
            _full_spec((IN_FEATS, HD)), _full_spec((1, HD)),
            _full_spec((1, HD)), _full_spec((1, HD)), _full_spec((1, 1)),
            _full_spec((HD, IN_FEATS)), _full_spec((1, IN_FEATS)),
            _full_spec((IN_FEATS, HD)), _full_spec((1, HD)),
            _full_spec((IN_FEATS, HD)), _full_spec((1, HD)),
            _full_spec((IN_FEATS, HD)), _full_spec((1, HD)),
            _full_spec((IN_FEATS, HD)), _full_spec((1, HD)),
        ],
        out_specs=[_row_spec(HD), _row_spec(2 * HD), _row_spec(HD)],
        out_shape=[
            jax.ShapeDtypeStruct((NPAD, HD), jnp.bfloat16),
            jax.ShapeDtypeStruct((NPAD, 2 * HD), jnp.bfloat16),
            jax.ShapeDtypeStruct((NPAD, HD), f32),
        ],
    )
    q0, kv0, skip0 = tc1(
        x_pad, oh_pad, tbl8,
        p['W_proj_num'], row1(p['b_proj_num']),
        p['W_proj_label'], row1(p['b_proj_label']),
        row1(p['bn_merge_g']), row1(p['bn_merge_b']), scl(p['a_merge']),
        p['W_merge'], row1(p['b_merge']),
        p['conv0_Wq'], row1(p['conv0_bq']),
        p['conv0_Wk'], row1(p['conv0_bk']),
        p['conv0_Wv'], row1(p['conv0_bv']),
        p['conv0_Wskip'], row1(p['conv0_bskip']),
    )

    acc0 = _edge_pass(_bitcast_pairs(q0), _bitcast_pairs(kv0), src_p, dst_p, zeros_blk)

    tc2 = pl.pallas_call(
        _tc2_body,
        grid=(GRID,),
        in_specs=[
            pl.BlockSpec((1, BR, ACC_W), lambda i: (0, i, 0)),
            pl.BlockSpec((1, BR, ACC_W), lambda i: (1, i, 0)),
            _row_spec(HD),
            _full_spec((3 * HD, 8)), _full_spec((1, 1)),
            _full_spec((HEADS, HD)),
            _full_spec((1, HD)), _full_spec((1, HD)), _full_spec((1, 1)),
            _full_spec((HD, HD)), _full_spec((1, HD)),
            _full_spec((HD, HD)), _full_spec((1, HD)),
            _full_spec((HD, HD)), _full_spec((1, HD)),
            _full_spec((HD, HD)), _full_spec((1, HD)),
        ],
        out_specs=[_row_spec(HD), _row_spec(2 * HD), _row_spec(HD)],
        out_shape=[
            jax.ShapeDtypeStruct((NPAD, HD), jnp.bfloat16),
            jax.ShapeDtypeStruct((NPAD, 2 * HD), jnp.bfloat16),
            jax.ShapeDtypeStruct((NPAD, HD), f32),
        ],
    )
    q1, kv1, skip1 = tc2(
        acc0, acc0, skip0,
        wg_pad, scl(p['conv0_bgate']), expand,
        row1(p['conv0_ln_g']), row1(p['conv0_ln_b']), scl(p['a_act']),
        p['conv1_Wq'], row1(p['conv1_bq']),
        p['conv1_Wk'], row1(p['conv1_bk']),
        p['conv1_Wv'], row1(p['conv1_bv']),
        p['conv1_Wskip'], row1(p['conv1_bskip']),
    )

    acc1 = _edge_pass(_bitcast_pairs(q1), _bitcast_pairs(kv1), src_p, dst_p, zeros_blk)

    tc3 = pl.pallas_call(
        _tc3_body,
        grid=(GRID,),
        in_specs=[
            pl.BlockSpec((1, BR, ACC_W), lambda i: (0, i, 0)),
            pl.BlockSpec((1, BR, ACC_W), lambda i: (1, i, 0)),
            _row_spec(HD),
            _full_spec((3 * HD, 8)), _full_spec((1, 1)),
            _full_spec((HEADS, HD)),
            _full_spec((1, HD)), _full_spec((1, HD)), _full_spec((1, 1)),
            _full_spec((HD, HD)), _full_spec((1, HD)),
            _full_spec((1, HD)), _full_spec((1, HD)), _full_spec((1, 1)),
            _full_spec((HD, 8)), _full_spec((1, 8)),
        ],
        out_specs=[_row_spec(8)],
        out_shape=[jax.ShapeDtypeStruct((NPAD, 8), f32)],
    )
    (z,) = tc3(
        acc1, acc1, skip1,
        wg1_pad, scl(p['conv1_bgate']), expand,
        row1(p['conv1_ln_g']), row1(p['conv1_ln_b']), scl(p['a_act']),
        p['W_clf1'], row1(p['b_clf1']),
        row1(p['bn_clf_g']), row1(p['bn_clf_b']), scl(p['a_clf']),
        w2_pad, b2_pad,
    )
    return z[:N, :N_CLASSES]
